# baseline (speedup 1.0000x reference)
import functools

import jax
import jax.numpy as jnp
from jax import lax
from jax.experimental import pallas as pl
from jax.experimental.pallas import tpu as pltpu
from jax.experimental.pallas import tpu_sc as plsc

_NB_ROWS = 4608
_CH = 96

_DN_T = (((1,), (1,)), ((), ()))


def _argmin_tail(dist, kdim, idx_ref):
    m = jnp.min(dist, axis=1, keepdims=True)
    ids = lax.broadcasted_iota(jnp.int32, dist.shape, 1).astype(jnp.float32)
    idx = jnp.min(jnp.where(dist == m, ids, float(kdim)), axis=1)
    idx_ref[0, 0, :] = idx.astype(jnp.int32)
    return jnp.sum(m)


def _stage0_body(e2_ref, x_ref, cb_ref, idx_ref, part_ref, *, kdim):
    x = x_ref[...]
    x2 = jnp.sum(x * x, axis=1, keepdims=True)
    xe2 = lax.dot_general(x * -2.0, cb_ref[...], _DN_T,
                          preferred_element_type=jnp.float32)
    dist = (x2 + e2_ref[...]) + xe2
    s = _argmin_tail(dist, kdim, idx_ref)
    i = pl.program_id(0)

    @pl.when(i == 0)
    def _():
        part_ref[0, 0] = s

    @pl.when(i != 0)
    def _():
        part_ref[0, 0] += s


def _stage1_body(e2_ref, x_ref, q0_ref, cb_ref, p0_ref, idx_ref, part_ref, *,
                 kdim, grid, scale):
    r = x_ref[...] - q0_ref[...]
    r2 = jnp.sum(r * r, axis=1, keepdims=True)
    xe2 = lax.dot_general(r * -2.0, cb_ref[...], _DN_T,
                          preferred_element_type=jnp.float32)
    dist = (r2 + e2_ref[...]) + xe2
    s = _argmin_tail(dist, kdim, idx_ref)
    i = pl.program_id(0)

    @pl.when(i == 0)
    def _():
        part_ref[0, 0] = s

    @pl.when(i != 0)
    def _():
        part_ref[0, 0] += s

    @pl.when(i == grid - 1)
    def _():
        part_ref[0, 0] = 1.25 * (part_ref[0, 0] + p0_ref[0, 0]) * scale


def _tc_stage(e2, x, q0, cb, p0):
    n, d = x.shape
    k = cb.shape[0]
    nb = _NB_ROWS
    grid = n // nb
    row_spec = pl.BlockSpec((nb, d), lambda i: (i, 0))
    smem_spec = pl.BlockSpec((1, 1), lambda i: (0, 0), memory_space=pltpu.SMEM)
    in_specs = [
        pl.BlockSpec((1, k), lambda i: (0, 0)),
        row_spec,
    ]
    args = [e2, x]
    if q0 is None:
        body = functools.partial(_stage0_body, kdim=k)
    else:
        body = functools.partial(_stage1_body, kdim=k, grid=grid,
                                 scale=1.0 / float(n * d))
        in_specs.append(row_spec)
        args.append(q0)
    in_specs.append(pl.BlockSpec((k, d), lambda i: (0, 0)))
    args.append(cb)
    if q0 is not None:
        in_specs.append(smem_spec)
        args.append(p0)
    idx, part = pl.pallas_call(
        body,
        grid=(grid,),
        in_specs=in_specs,
        out_specs=[
            pl.BlockSpec((1, 1, nb), lambda i: (i, 0, 0)),
            smem_spec,
        ],
        out_shape=[
            jax.ShapeDtypeStruct((grid, 1, nb), jnp.int32),
            jax.ShapeDtypeStruct((1, 1), jnp.float32),
        ],
    )(*args)
    return idx.reshape(n), part


def _sc_gather(cb, idx):
    info = plsc.get_sparse_core_info()
    ncores, nsub = info.num_cores, info.num_subcores
    nw = ncores * nsub
    n = idx.shape[0]
    d = cb.shape[1]
    rows_w = n // nw
    ch = _CH
    nch = rows_w // ch
    mesh = plsc.VectorSubcoreMesh(core_axis_name="c", subcore_axis_name="s")

    @functools.partial(
        pl.kernel,
        out_type=jax.ShapeDtypeStruct((n, d), jnp.float32),
        mesh=mesh,
        scratch_types=[
            pltpu.VMEM((ch,), jnp.int32),
            pltpu.VMEM((ch, d), jnp.float32),
            pltpu.SemaphoreType.DMA,
        ],
    )
    def k(cb_hbm, idx_hbm, out_hbm, idx_v, rows_v, sem):
        wid = lax.axis_index("s") * ncores + lax.axis_index("c")
        base = wid * rows_w
        for c in range(nch):
            off = base + c * ch
            pltpu.sync_copy(idx_hbm.at[pl.ds(off, ch)], idx_v)
            pltpu.async_copy(cb_hbm.at[idx_v], rows_v, sem).wait()
            pltpu.sync_copy(rows_v, out_hbm.at[pl.ds(off, ch)])

    return k(cb, idx)


def _sc_gather_add(cb, idx, prev):
    info = plsc.get_sparse_core_info()
    ncores, nsub = info.num_cores, info.num_subcores
    nw = ncores * nsub
    n = idx.shape[0]
    d = cb.shape[1]
    rows_w = n // nw
    ch = _CH
    nch = rows_w // ch
    mesh = plsc.VectorSubcoreMesh(core_axis_name="c", subcore_axis_name="s")

    @functools.partial(
        pl.kernel,
        out_type=jax.ShapeDtypeStruct((n, d), jnp.float32),
        mesh=mesh,
        scratch_types=[
            pltpu.VMEM((ch,), jnp.int32),
            pltpu.VMEM((ch, d), jnp.float32),
            pltpu.VMEM((ch, d), jnp.float32),
            pltpu.SemaphoreType.DMA,
        ],
    )
    def k(cb_hbm, idx_hbm, prev_hbm, out_hbm, idx_v, rows_v, acc_v, sem):
        wid = lax.axis_index("s") * ncores + lax.axis_index("c")
        base = wid * rows_w
        for c in range(nch):
            off = base + c * ch
            pltpu.sync_copy(idx_hbm.at[pl.ds(off, ch)], idx_v)
            cp = pltpu.async_copy(cb_hbm.at[idx_v], rows_v, sem)
            pltpu.sync_copy(prev_hbm.at[pl.ds(off, ch)], acc_v)
            cp.wait()

            def body(rr, carry):
                for j in range(d // 16):
                    sl = pl.ds(j * 16, 16)
                    plsc.addupdate(acc_v.at[rr, sl], rows_v[rr, sl])
                return carry

            lax.fori_loop(0, ch, body, 0)
            pltpu.sync_copy(acc_v, out_hbm.at[pl.ds(off, ch)])

    return k(cb, idx, prev)


def kernel(x, cb0, cb1):
    b, t, d = x.shape
    n = b * t
    xf = x.reshape(n, d)

    e2_0 = (cb0 ** 2).sum(axis=1)[None, :]
    idx0, part0 = _tc_stage(e2_0, xf, None, cb0, None)

    q0 = _sc_gather(cb0, idx0)

    e2_1 = (cb1 ** 2).sum(axis=1)[None, :]
    idx1, loss = _tc_stage(e2_1, xf, q0, cb1, part0)

    qt = _sc_gather_add(cb1, idx1, q0)

    quantized = qt.reshape(b, t, d)
    codes = jnp.stack([idx0.reshape(b, t), idx1.reshape(b, t)], axis=0)
    return quantized, codes, loss.reshape(())


# speedup vs baseline: 1.0117x; 1.0117x over previous
import functools

import jax
import jax.numpy as jnp
from jax import lax
from jax.experimental import pallas as pl
from jax.experimental.pallas import tpu as pltpu
from jax.experimental.pallas import tpu_sc as plsc

_NB_ROWS = 2304
_CH = 96

_DN_T = (((1,), (1,)), ((), ()))


def _argmin_tail(dist, kdim, idx_ref):
    m = jnp.min(dist, axis=1, keepdims=True)
    ids = lax.broadcasted_iota(jnp.int32, dist.shape, 1).astype(jnp.float32)
    idx = jnp.min(jnp.where(dist == m, ids, float(kdim)), axis=1)
    idx_ref[0, 0, :] = idx.astype(jnp.int32)
    return jnp.sum(m)


def _stage0_body(e2_ref, x_ref, cb_ref, idx_ref, part_ref, *, kdim):
    x = x_ref[...]
    x2 = jnp.sum(x * x, axis=1, keepdims=True)
    xe2 = lax.dot_general(x * -2.0, cb_ref[...], _DN_T,
                          preferred_element_type=jnp.float32)
    dist = (x2 + e2_ref[...]) + xe2
    s = _argmin_tail(dist, kdim, idx_ref)
    i = pl.program_id(0)

    @pl.when(i == 0)
    def _():
        part_ref[0, 0] = s

    @pl.when(i != 0)
    def _():
        part_ref[0, 0] += s


def _stage1_body(e2_ref, x_ref, q0_ref, cb_ref, p0_ref, idx_ref, part_ref, *,
                 kdim, grid, scale):
    r = x_ref[...] - q0_ref[...]
    r2 = jnp.sum(r * r, axis=1, keepdims=True)
    xe2 = lax.dot_general(r * -2.0, cb_ref[...], _DN_T,
                          preferred_element_type=jnp.float32)
    dist = (r2 + e2_ref[...]) + xe2
    s = _argmin_tail(dist, kdim, idx_ref)
    i = pl.program_id(0)

    @pl.when(i == 0)
    def _():
        part_ref[0, 0] = s

    @pl.when(i != 0)
    def _():
        part_ref[0, 0] += s

    @pl.when(i == grid - 1)
    def _():
        part_ref[0, 0] = 1.25 * (part_ref[0, 0] + p0_ref[0, 0]) * scale


def _tc_stage(e2, x, q0, cb, p0):
    n, d = x.shape
    k = cb.shape[0]
    nb = _NB_ROWS
    grid = n // nb
    row_spec = pl.BlockSpec((nb, d), lambda i: (i, 0))
    smem_spec = pl.BlockSpec((1, 1), lambda i: (0, 0), memory_space=pltpu.SMEM)
    in_specs = [
        pl.BlockSpec((1, k), lambda i: (0, 0)),
        row_spec,
    ]
    args = [e2, x]
    if q0 is None:
        body = functools.partial(_stage0_body, kdim=k)
    else:
        body = functools.partial(_stage1_body, kdim=k, grid=grid,
                                 scale=1.0 / float(n * d))
        in_specs.append(row_spec)
        args.append(q0)
    in_specs.append(pl.BlockSpec((k, d), lambda i: (0, 0)))
    args.append(cb)
    if q0 is not None:
        in_specs.append(smem_spec)
        args.append(p0)
    idx, part = pl.pallas_call(
        body,
        grid=(grid,),
        in_specs=in_specs,
        out_specs=[
            pl.BlockSpec((1, 1, nb), lambda i: (i, 0, 0)),
            smem_spec,
        ],
        out_shape=[
            jax.ShapeDtypeStruct((grid, 1, nb), jnp.int32),
            jax.ShapeDtypeStruct((1, 1), jnp.float32),
        ],
    )(*args)
    return idx.reshape(n), part


def _sc_gather(cb, idx):
    info = plsc.get_sparse_core_info()
    ncores, nsub = info.num_cores, info.num_subcores
    nw = ncores * nsub
    n = idx.shape[0]
    d = cb.shape[1]
    rows_w = n // nw
    ch = _CH
    nch = rows_w // ch
    mesh = plsc.VectorSubcoreMesh(core_axis_name="c", subcore_axis_name="s")

    @functools.partial(
        pl.kernel,
        out_type=jax.ShapeDtypeStruct((n, d), jnp.float32),
        mesh=mesh,
        scratch_types=[
            pltpu.VMEM((ch,), jnp.int32),
            pltpu.VMEM((ch, d), jnp.float32),
            pltpu.SemaphoreType.DMA,
        ],
    )
    def k(cb_hbm, idx_hbm, out_hbm, idx_v, rows_v, sem):
        wid = lax.axis_index("s") * ncores + lax.axis_index("c")
        base = wid * rows_w
        for c in range(nch):
            off = base + c * ch
            pltpu.sync_copy(idx_hbm.at[pl.ds(off, ch)], idx_v)
            pltpu.async_copy(cb_hbm.at[idx_v], rows_v, sem).wait()
            pltpu.sync_copy(rows_v, out_hbm.at[pl.ds(off, ch)])

    return k(cb, idx)


def _sc_gather_add(cb, idx, prev):
    info = plsc.get_sparse_core_info()
    ncores, nsub = info.num_cores, info.num_subcores
    nw = ncores * nsub
    n = idx.shape[0]
    d = cb.shape[1]
    rows_w = n // nw
    ch = _CH
    nch = rows_w // ch
    mesh = plsc.VectorSubcoreMesh(core_axis_name="c", subcore_axis_name="s")

    @functools.partial(
        pl.kernel,
        out_type=jax.ShapeDtypeStruct((n, d), jnp.float32),
        mesh=mesh,
        scratch_types=[
            pltpu.VMEM((ch,), jnp.int32),
            pltpu.VMEM((ch, d), jnp.float32),
            pltpu.VMEM((ch, d), jnp.float32),
            pltpu.SemaphoreType.DMA,
        ],
    )
    def k(cb_hbm, idx_hbm, prev_hbm, out_hbm, idx_v, rows_v, acc_v, sem):
        wid = lax.axis_index("s") * ncores + lax.axis_index("c")
        base = wid * rows_w
        for c in range(nch):
            off = base + c * ch
            pltpu.sync_copy(idx_hbm.at[pl.ds(off, ch)], idx_v)
            cp = pltpu.async_copy(cb_hbm.at[idx_v], rows_v, sem)
            pltpu.sync_copy(prev_hbm.at[pl.ds(off, ch)], acc_v)
            cp.wait()

            def body(rr, carry):
                for j in range(d // 16):
                    sl = pl.ds(j * 16, 16)
                    plsc.addupdate(acc_v.at[rr, sl], rows_v[rr, sl])
                return carry

            lax.fori_loop(0, ch, body, 0)
            pltpu.sync_copy(acc_v, out_hbm.at[pl.ds(off, ch)])

    return k(cb, idx, prev)


def kernel(x, cb0, cb1):
    b, t, d = x.shape
    n = b * t
    xf = x.reshape(n, d)

    e2_0 = (cb0 ** 2).sum(axis=1)[None, :]
    idx0, part0 = _tc_stage(e2_0, xf, None, cb0, None)

    q0 = _sc_gather(cb0, idx0)

    e2_1 = (cb1 ** 2).sum(axis=1)[None, :]
    idx1, loss = _tc_stage(e2_1, xf, q0, cb1, part0)

    qt = _sc_gather_add(cb1, idx1, q0)

    quantized = qt.reshape(b, t, d)
    codes = jnp.stack([idx0.reshape(b, t), idx1.reshape(b, t)], axis=0)
    return quantized, codes, loss.reshape(())
